# baseline (speedup 1.0000x reference)
import functools

import jax
import jax.numpy as jnp
from jax import lax
from jax.experimental import pallas as pl
from jax.experimental.pallas import tpu as pltpu
from jax.experimental.pallas import tpu_sc as plsc

MAX_THETA = 90.0
MAX_RADIUS = 300.0
THR_T = 3.0 / MAX_THETA
THR_R = 10.0 / MAX_RADIUS
W_CLS = 2.0
W_REG = 1.0

B, N, G = 16, 1000, 32
NP = 1024
NCHUNK = NP // 16
UNROLL = 8
GH = G // 2
PAD_INIT = 2.0e9


def _dyn_gather(vec, idx):
    dnums = lax.GatherDimensionNumbers(
        offset_dims=(), collapsed_slice_dims=(0,), start_index_map=(0,))
    return lax.gather(vec, idx[:, None], dnums, slice_sizes=(1,),
                      mode=lax.GatherScatterMode.PROMISE_IN_BOUNDS)


def _bfly(v, op):
    lane = lax.iota(jnp.int32, 16)
    for s in (8, 4, 2, 1):
        v = op(v, _dyn_gather(v, lane ^ s))
    return v


def _sc_match_kernel(big_h, small_h, gt_h, lossp_h,
                     raw_v, small_v, init0_v, init1_v,
                     par0_v, par1_v, tgt0_v, tgt1_v, valid_v, gtrow_v,
                     out16_v):
    cid = lax.axis_index("c")
    sid = lax.axis_index("s")
    wid = sid * 2 + cid
    b = wid // 2
    half = wid % 2
    g0 = half * GH

    lane = lax.iota(jnp.int32, 16)
    lane_f = lane.astype(jnp.float32)
    zf = jnp.zeros((16,), jnp.float32)
    ev = (lane % 8) * 2
    od = ev + 1

    def _deint(a_vec, b_vec):
        ta = _dyn_gather(a_vec, ev)
        tb = _dyn_gather(b_vec, ev)
        ra = _dyn_gather(a_vec, od)
        rb = _dyn_gather(b_vec, od)
        lo = lane < 8
        return jnp.where(lo, ta, tb), jnp.where(lo, ra, rb)

    pad_v = jnp.full((16,), PAD_INIT, jnp.float32)

    def _deint_row(dst0, dst1, tail_pad):
        def body(c, carry):
            a_vec = raw_v[pl.ds(c * 32, 16)]
            b_vec = raw_v[pl.ds(c * 32 + 16, 16)]
            t, r = _deint(a_vec, b_vec)
            dst0[pl.ds(c * 16, 16)] = t
            dst1[pl.ds(c * 16, 16)] = r
            return carry
        lax.fori_loop(0, (2 * N) // 32, body, 0)
        a_vec = raw_v[pl.ds(2 * N - 16, 16)]
        t, r = _deint(a_vec, a_vec)
        lo = lane < 8
        dst0[pl.ds(N - 8, 16)] = jnp.where(lo, t, tail_pad)
        dst1[pl.ds(N - 8, 16)] = jnp.where(lo, r, tail_pad)
        dst0[pl.ds(N + 8, 16)] = tail_pad
        dst1[pl.ds(N + 8, 16)] = tail_pad

    pltpu.sync_copy(big_h.at[2 * b], raw_v)
    _deint_row(init0_v, init1_v, pad_v)
    pltpu.sync_copy(big_h.at[2 * b + 1], raw_v)
    _deint_row(par0_v, par1_v, pad_v)

    def zero_body(c, carry):
        gtrow_v[pl.ds(c * 16, 16)] = zf
        return carry
    lax.fori_loop(0, NCHUNK, zero_body, 0)

    pltpu.sync_copy(small_h.at[2 * b], small_v)
    t, r = _deint(small_v[pl.ds(g0 * 2, 16)], small_v[pl.ds(g0 * 2 + 16, 16)])
    tgt0_v[...] = t
    tgt1_v[...] = r
    pltpu.sync_copy(small_h.at[2 * b + 1], small_v)
    pv, _ = _deint(small_v[pl.ds(g0 * 2, 16)], small_v[pl.ds(g0 * 2 + 16, 16)])
    valid_v[...] = jnp.where(pv != -1000.0, 1.0, 0.0)

    def gt_body(g, loss_acc):
        gidx = jnp.broadcast_to(g, (16,)).astype(jnp.int32)
        t0r = _dyn_gather(tgt0_v[...], gidx)
        t1r = _dyn_gather(tgt1_v[...], gidx)
        vgf = _dyn_gather(valid_v[...], gidx)
        t0 = (t0r + MAX_THETA) / (2.0 * MAX_THETA)
        t1 = (t1r + MAX_RADIUS) / (2.0 * MAX_RADIUS)

        def chunk_body(cc, carry):
            min_d2, min_idx, cost_sum, anyacc, nidx = carry
            for u in range(UNROLL):
                sl = pl.ds((cc * UNROLL + u) * 16, 16)
                i0 = init0_v[sl]
                i1 = init1_v[sl]
                td = jnp.abs(t0 - i0)
                rd = jnp.abs(t1 - i1)
                cond = jnp.logical_and(td < THR_T, rd < THR_R)
                d2 = rd * rd + td * td
                p0 = par0_v[sl]
                p1 = par1_v[sl]
                dt = t0 - p0
                dr = t1 - p1
                cost = dt * dt + dr * dr
                upd = d2 < min_d2
                min_d2 = jnp.where(upd, d2, min_d2)
                min_idx = jnp.where(upd, nidx, min_idx)
                cost_sum = cost_sum + jnp.where(cond, cost, 0.0)
                bits = jnp.where(cond, vgf, 0.0)
                gtrow_v[sl] = jnp.maximum(gtrow_v[sl], bits)
                anyacc = jnp.maximum(anyacc, bits)
                nidx = nidx + 16.0
            return (min_d2, min_idx, cost_sum, anyacc, nidx)

        init_carry = (jnp.full((16,), 3.0e38, jnp.float32), zf, zf, zf, lane_f)
        min_d2, min_idx, cost_sum, anyacc, _ = lax.fori_loop(
            0, NCHUNK // UNROLL, chunk_body, init_carry)

        has_pos = _bfly(anyacc, jnp.maximum)
        gmin = _bfly(min_d2, jnp.minimum)
        cand = jnp.where(min_d2 == gmin, min_idx, 1.0e9)
        nfb_f = _bfly(cand, jnp.minimum)
        nfb = nfb_f.astype(jnp.int32)[0]
        cfb = nfb // 16
        lfb = nfb % 16
        slf = pl.ds(cfb * 16, 16)

        lfb_v = jnp.broadcast_to(lfb, (16,))
        pf0 = _dyn_gather(par0_v[slf], lfb_v)
        pf1 = _dyn_gather(par1_v[slf], lfb_v)
        df0 = t0 - pf0
        df1 = t1 - pf1
        cost_fb = df0 * df0 + df1 * df1
        fb_contrib = jnp.where(lane == 0, cost_fb, 0.0)
        contrib = jnp.where(has_pos > 0.5, cost_sum, fb_contrib)
        loss_acc = loss_acc + vgf * contrib

        fbbits = jnp.where(lane == lfb, (1.0 - has_pos) * vgf, 0.0)
        gtrow_v[slf] = jnp.maximum(gtrow_v[slf], fbbits)
        return loss_acc

    loss_acc = lax.fori_loop(0, GH, gt_body, zf)
    loss_w = _bfly(loss_acc, lambda a, c: a + c) * 0.5
    out16_v[...] = jnp.where(lane == 0, loss_w, 0.0)
    pltpu.sync_copy(out16_v, lossp_h.at[wid])
    pltpu.sync_copy(gtrow_v, gt_h.at[b, half])


def _make_sc_call():
    mesh = plsc.VectorSubcoreMesh(core_axis_name="c", subcore_axis_name="s")
    return functools.partial(
        pl.kernel,
        out_type=[
            jax.ShapeDtypeStruct((B, 2, NP), jnp.float32),
            jax.ShapeDtypeStruct((2 * B, 16), jnp.float32),
        ],
        mesh=mesh,
        scratch_types=[
            pltpu.VMEM((2 * N,), jnp.float32),
            pltpu.VMEM((2 * G,), jnp.float32),
            pltpu.VMEM((NP,), jnp.float32),
            pltpu.VMEM((NP,), jnp.float32),
            pltpu.VMEM((NP,), jnp.float32),
            pltpu.VMEM((NP,), jnp.float32),
            pltpu.VMEM((GH,), jnp.float32),
            pltpu.VMEM((GH,), jnp.float32),
            pltpu.VMEM((GH,), jnp.float32),
            pltpu.VMEM((NP,), jnp.float32),
            pltpu.VMEM((16,), jnp.float32),
        ],
    )(_sc_match_kernel)


def _tc_focal_kernel(l0_ref, l1_ref, h_ref, lossp_ref, cls_out, reg_out):
    l0 = l0_ref[...]
    l1 = l1_ref[...]
    h = h_ref[...]
    gt = (h[:, 0, :N] + h[:, 1, :N]) > 0.5
    m = jnp.maximum(l0, l1)
    e0 = jnp.exp(l0 - m)
    e1 = jnp.exp(l1 - m)
    z = e0 + e1
    logz = jnp.log(z) + m
    lp0 = l0 - logz
    lp1 = l1 - logz
    p0 = e0 / z
    p1 = e1 / z
    pick_lp = jnp.where(gt, lp1, lp0)
    pick_p = jnp.where(gt, p1, p0)
    om = 1.0 - pick_p
    focal = om * om * pick_lp
    loss_cls = -(jnp.sum(focal) / (B * N))
    loss_reg = jnp.sum(lossp_ref[...]) / B
    cls_out[...] = jnp.reshape(loss_cls * W_CLS, (1, 1))
    reg_out[...] = jnp.reshape(loss_reg * W_REG, (1, 1))


@jax.jit
def kernel(cls_logits, params, params_init, tgt_params, pts):
    big = jnp.stack([params_init.reshape(B, 2 * N),
                     params.reshape(B, 2 * N)], axis=1).reshape(2 * B, 2 * N)
    small = jnp.stack([tgt_params.reshape(B, 2 * G),
                       pts.reshape(B, 2 * G)], axis=1).reshape(2 * B, 2 * G)

    gt_halves, lossp = _make_sc_call()(big, small)

    cls_l, reg_l = pl.pallas_call(
        _tc_focal_kernel,
        out_shape=[
            jax.ShapeDtypeStruct((1, 1), jnp.float32),
            jax.ShapeDtypeStruct((1, 1), jnp.float32),
        ],
    )(cls_logits[..., 0], cls_logits[..., 1], gt_halves, lossp)
    return (cls_l[0, 0], reg_l[0, 0])


# speedup vs baseline: 1.0273x; 1.0273x over previous
import functools

import jax
import jax.numpy as jnp
from jax import lax
from jax.experimental import pallas as pl
from jax.experimental.pallas import tpu as pltpu
from jax.experimental.pallas import tpu_sc as plsc

MAX_THETA = 90.0
MAX_RADIUS = 300.0
THR_T = 3.0 / MAX_THETA
THR_R = 10.0 / MAX_RADIUS
W_CLS = 2.0
W_REG = 1.0

B, N, G = 16, 1000, 32
NP = 1024
NCHUNK = NP // 16
UNROLL = 4
GH = G // 2
PAD_INIT = 2.0e9


def _dyn_gather(vec, idx):
    dnums = lax.GatherDimensionNumbers(
        offset_dims=(), collapsed_slice_dims=(0,), start_index_map=(0,))
    return lax.gather(vec, idx[:, None], dnums, slice_sizes=(1,),
                      mode=lax.GatherScatterMode.PROMISE_IN_BOUNDS)


def _bfly(v, op):
    lane = lax.iota(jnp.int32, 16)
    for s in (8, 4, 2, 1):
        v = op(v, _dyn_gather(v, lane ^ s))
    return v


def _sc_match_kernel(big_h, small_h, gt_h, lossp_h,
                     raw_v, small_v, init0_v, init1_v,
                     par0_v, par1_v, tgt0_v, tgt1_v, valid_v, gtrow_v,
                     out16_v):
    cid = lax.axis_index("c")
    sid = lax.axis_index("s")
    wid = sid * 2 + cid
    b = wid // 2
    half = wid % 2
    g0 = half * GH

    lane = lax.iota(jnp.int32, 16)
    lane_f = lane.astype(jnp.float32)
    zf = jnp.zeros((16,), jnp.float32)
    ev = (lane % 8) * 2
    od = ev + 1

    def _deint(a_vec, b_vec):
        ta = _dyn_gather(a_vec, ev)
        tb = _dyn_gather(b_vec, ev)
        ra = _dyn_gather(a_vec, od)
        rb = _dyn_gather(b_vec, od)
        lo = lane < 8
        return jnp.where(lo, ta, tb), jnp.where(lo, ra, rb)

    pad_v = jnp.full((16,), PAD_INIT, jnp.float32)

    def _deint_row(dst0, dst1, tail_pad):
        def body(c, carry):
            a_vec = raw_v[pl.ds(c * 32, 16)]
            b_vec = raw_v[pl.ds(c * 32 + 16, 16)]
            t, r = _deint(a_vec, b_vec)
            dst0[pl.ds(c * 16, 16)] = t
            dst1[pl.ds(c * 16, 16)] = r
            return carry
        lax.fori_loop(0, (2 * N) // 32, body, 0)
        a_vec = raw_v[pl.ds(2 * N - 16, 16)]
        t, r = _deint(a_vec, a_vec)
        lo = lane < 8
        dst0[pl.ds(N - 8, 16)] = jnp.where(lo, t, tail_pad)
        dst1[pl.ds(N - 8, 16)] = jnp.where(lo, r, tail_pad)
        dst0[pl.ds(N + 8, 16)] = tail_pad
        dst1[pl.ds(N + 8, 16)] = tail_pad

    pltpu.sync_copy(big_h.at[2 * b], raw_v)
    _deint_row(init0_v, init1_v, pad_v)
    pltpu.sync_copy(big_h.at[2 * b + 1], raw_v)
    _deint_row(par0_v, par1_v, pad_v)

    def zero_body(c, carry):
        gtrow_v[pl.ds(c * 16, 16)] = zf
        return carry
    lax.fori_loop(0, NCHUNK, zero_body, 0)

    pltpu.sync_copy(small_h.at[2 * b], small_v)
    t, r = _deint(small_v[pl.ds(g0 * 2, 16)], small_v[pl.ds(g0 * 2 + 16, 16)])
    tgt0_v[...] = t
    tgt1_v[...] = r
    pltpu.sync_copy(small_h.at[2 * b + 1], small_v)
    pv, _ = _deint(small_v[pl.ds(g0 * 2, 16)], small_v[pl.ds(g0 * 2 + 16, 16)])
    valid_v[...] = jnp.where(pv != -1000.0, 1.0, 0.0)

    def gt_body(g, loss_acc):
        gidx = jnp.broadcast_to(g, (16,)).astype(jnp.int32)
        t0r = _dyn_gather(tgt0_v[...], gidx)
        t1r = _dyn_gather(tgt1_v[...], gidx)
        vgf = _dyn_gather(valid_v[...], gidx)
        t0 = (t0r + MAX_THETA) / (2.0 * MAX_THETA)
        t1 = (t1r + MAX_RADIUS) / (2.0 * MAX_RADIUS)

        def chunk_body(cc, carry):
            min_d2, min_idx, cost_sum, anyacc, nidx = carry
            for u in range(UNROLL):
                sl = pl.ds((cc * UNROLL + u) * 16, 16)
                i0 = init0_v[sl]
                i1 = init1_v[sl]
                td = jnp.abs(t0 - i0)
                rd = jnp.abs(t1 - i1)
                cond = jnp.logical_and(td < THR_T, rd < THR_R)
                d2 = rd * rd + td * td
                p0 = par0_v[sl]
                p1 = par1_v[sl]
                dt = t0 - p0
                dr = t1 - p1
                cost = dt * dt + dr * dr
                upd = d2 < min_d2
                min_d2 = jnp.where(upd, d2, min_d2)
                min_idx = jnp.where(upd, nidx, min_idx)
                cost_sum = cost_sum + jnp.where(cond, cost, 0.0)
                bits = jnp.where(cond, vgf, 0.0)
                gtrow_v[sl] = jnp.maximum(gtrow_v[sl], bits)
                anyacc = jnp.maximum(anyacc, bits)
                nidx = nidx + 16.0
            return (min_d2, min_idx, cost_sum, anyacc, nidx)

        init_carry = (jnp.full((16,), 3.0e38, jnp.float32), zf, zf, zf, lane_f)
        min_d2, min_idx, cost_sum, anyacc, _ = lax.fori_loop(
            0, NCHUNK // UNROLL, chunk_body, init_carry)

        has_pos = _bfly(anyacc, jnp.maximum)
        gmin = _bfly(min_d2, jnp.minimum)
        cand = jnp.where(min_d2 == gmin, min_idx, 1.0e9)
        nfb_f = _bfly(cand, jnp.minimum)
        nfb = nfb_f.astype(jnp.int32)[0]
        cfb = nfb // 16
        lfb = nfb % 16
        slf = pl.ds(cfb * 16, 16)

        lfb_v = jnp.broadcast_to(lfb, (16,))
        pf0 = _dyn_gather(par0_v[slf], lfb_v)
        pf1 = _dyn_gather(par1_v[slf], lfb_v)
        df0 = t0 - pf0
        df1 = t1 - pf1
        cost_fb = df0 * df0 + df1 * df1
        fb_contrib = jnp.where(lane == 0, cost_fb, 0.0)
        contrib = jnp.where(has_pos > 0.5, cost_sum, fb_contrib)
        loss_acc = loss_acc + vgf * contrib

        fbbits = jnp.where(lane == lfb, (1.0 - has_pos) * vgf, 0.0)
        gtrow_v[slf] = jnp.maximum(gtrow_v[slf], fbbits)
        return loss_acc

    loss_acc = lax.fori_loop(0, GH, gt_body, zf)
    loss_w = _bfly(loss_acc, lambda a, c: a + c) * 0.5
    out16_v[...] = jnp.where(lane == 0, loss_w, 0.0)
    pltpu.sync_copy(out16_v, lossp_h.at[wid])
    pltpu.sync_copy(gtrow_v, gt_h.at[b, half])


def _make_sc_call():
    mesh = plsc.VectorSubcoreMesh(core_axis_name="c", subcore_axis_name="s")
    return functools.partial(
        pl.kernel,
        out_type=[
            jax.ShapeDtypeStruct((B, 2, NP), jnp.float32),
            jax.ShapeDtypeStruct((2 * B, 16), jnp.float32),
        ],
        mesh=mesh,
        scratch_types=[
            pltpu.VMEM((2 * N,), jnp.float32),
            pltpu.VMEM((2 * G,), jnp.float32),
            pltpu.VMEM((NP,), jnp.float32),
            pltpu.VMEM((NP,), jnp.float32),
            pltpu.VMEM((NP,), jnp.float32),
            pltpu.VMEM((NP,), jnp.float32),
            pltpu.VMEM((GH,), jnp.float32),
            pltpu.VMEM((GH,), jnp.float32),
            pltpu.VMEM((GH,), jnp.float32),
            pltpu.VMEM((NP,), jnp.float32),
            pltpu.VMEM((16,), jnp.float32),
        ],
    )(_sc_match_kernel)


def _tc_focal_kernel(l0_ref, l1_ref, h_ref, lossp_ref, cls_out, reg_out):
    l0 = l0_ref[...]
    l1 = l1_ref[...]
    h = h_ref[...]
    gt = (h[:, 0, :N] + h[:, 1, :N]) > 0.5
    m = jnp.maximum(l0, l1)
    e0 = jnp.exp(l0 - m)
    e1 = jnp.exp(l1 - m)
    z = e0 + e1
    logz = jnp.log(z) + m
    lp0 = l0 - logz
    lp1 = l1 - logz
    p0 = e0 / z
    p1 = e1 / z
    pick_lp = jnp.where(gt, lp1, lp0)
    pick_p = jnp.where(gt, p1, p0)
    om = 1.0 - pick_p
    focal = om * om * pick_lp
    loss_cls = -(jnp.sum(focal) / (B * N))
    loss_reg = jnp.sum(lossp_ref[...]) / B
    cls_out[...] = jnp.reshape(loss_cls * W_CLS, (1, 1))
    reg_out[...] = jnp.reshape(loss_reg * W_REG, (1, 1))


@jax.jit
def kernel(cls_logits, params, params_init, tgt_params, pts):
    big = jnp.stack([params_init.reshape(B, 2 * N),
                     params.reshape(B, 2 * N)], axis=1).reshape(2 * B, 2 * N)
    small = jnp.stack([tgt_params.reshape(B, 2 * G),
                       pts.reshape(B, 2 * G)], axis=1).reshape(2 * B, 2 * G)

    gt_halves, lossp = _make_sc_call()(big, small)

    cls_l, reg_l = pl.pallas_call(
        _tc_focal_kernel,
        out_shape=[
            jax.ShapeDtypeStruct((1, 1), jnp.float32),
            jax.ShapeDtypeStruct((1, 1), jnp.float32),
        ],
    )(cls_logits[..., 0], cls_logits[..., 1], gt_halves, lossp)
    return (cls_l[0, 0], reg_l[0, 0])
